# Initial kernel scaffold; baseline (speedup 1.0000x reference)
import jax
import jax.numpy as jnp
from jax.experimental import pallas as pl


def kernel(x, pos, Wc0, bc0, g0, be0, Wc1, bc1, g1, be1, Wc2, bc2, g2, be2, W1, b1, W2, b2):
    raise NotImplementedError("write your pallas kernel here")


# speedup vs baseline: 22.7411x; 22.7411x over previous
import functools

import jax
import jax.numpy as jnp
from jax import lax
from jax.experimental import pallas as pl
from jax.experimental.pallas import tpu as pltpu
from jax.experimental.pallas import tpu_sc as plsc

B, C, H, W = 4, 192, 56, 56
N = H * W
BN = B * N
K = 5
CP = 256

KNN_BLK = 392
KNN_NBLK = N // KNN_BLK

SC_W = 32
NPW = BN // SC_W
CHN = 56
NCH = NPW // CHN

M_BLK = 1568
M_NBLK = BN // M_BLK


def _knn_body(coords_ref, coordsT_ref, out_ref):
    b = pl.program_id(0)
    cb = coords_ref[0]
    ct = coordsT_ref[0]
    xi = cb[:, 0:1]
    yi = cb[:, 1:2]
    xj = ct[0:1, :]
    yj = ct[1:2, :]
    dot = jnp.dot(cb, ct, preferred_element_type=jnp.float32)
    sq_i = xi * xi + yi * yi
    sq_j = xj * xj + yj * yj
    d = (sq_i + sq_j) - 2.0 * dot
    lane = lax.broadcasted_iota(jnp.int32, (KNN_BLK, N), 1)
    cols = []
    for _ in range(K):
        mn = jnp.min(d, axis=1, keepdims=True)
        idx = jnp.min(jnp.where(d == mn, lane, N), axis=1, keepdims=True)
        cols.append(idx + b * N)
        d = jnp.where(lane == idx, jnp.inf, d)
    cols.append(jnp.zeros((KNN_BLK, 3), jnp.int32))
    out_ref[0] = jnp.concatenate(cols, axis=1)


def _knn(coords, coordsT):
    return pl.pallas_call(
        _knn_body,
        grid=(B, KNN_NBLK),
        in_specs=[
            pl.BlockSpec((1, KNN_BLK, 2), lambda b, j: (b, j, 0)),
            pl.BlockSpec((1, 2, N), lambda b, j: (b, 0, 0)),
        ],
        out_specs=pl.BlockSpec((1, KNN_BLK, 8), lambda b, j: (b, j, 0)),
        out_shape=jax.ShapeDtypeStruct((B, N, 8), jnp.int32),
    )(coords, coordsT)


def _gather_max(h, gidx):
    mesh = plsc.VectorSubcoreMesh(core_axis_name="c", subcore_axis_name="s")

    @functools.partial(
        pl.kernel,
        out_type=jax.ShapeDtypeStruct((BN, CP), jnp.float32),
        mesh=mesh,
        scratch_types=[
            pltpu.VMEM((K, CHN, CP), jnp.float32),
            pltpu.VMEM((K, CHN), jnp.int32),
            pltpu.VMEM((CHN, CP), jnp.float32),
            pltpu.SemaphoreType.DMA,
        ],
    )
    def run(h_hbm, idx_hbm, out_hbm, rows_v, idx_v, out_v, sem):
        wid = lax.axis_index("s") * 2 + lax.axis_index("c")
        for ch in range(NCH):
            pltpu.sync_copy(idx_hbm.at[wid, ch], idx_v)
            cps = [
                pltpu.async_copy(h_hbm.at[idx_v.at[k]], rows_v.at[k], sem)
                for k in range(K)
            ]
            for cp in cps:
                cp.wait()

            def body(n, carry):
                for c0 in range(CP // 16):
                    sl = pl.ds(c0 * 16, 16)
                    m = rows_v[0, n, sl]
                    for k in range(1, K):
                        m = jnp.maximum(m, rows_v[k, n, sl])
                    out_v[n, sl] = m
                return carry

            lax.fori_loop(0, CHN, body, 0)
            pltpu.sync_copy(
                out_v, out_hbm.at[pl.ds(wid * NPW + ch * CHN, CHN)]
            )

    return run(h, gidx)


def _layer_body(h_ref, mx_ref, A_ref, Wo_ref, bc_ref, g_ref, be_ref,
                out_ref, zbuf, ssum, ssq):
    p = pl.program_id(0)
    j = pl.program_id(1)
    rows = pl.ds(j * M_BLK, M_BLK)

    @pl.when((p == 0) & (j == 0))
    def _init():
        ssum[...] = jnp.zeros_like(ssum)
        ssq[...] = jnp.zeros_like(ssq)

    @pl.when(p == 0)
    def _compute():
        z = (jnp.dot(h_ref[...], A_ref[...], precision=lax.Precision.HIGHEST,
                     preferred_element_type=jnp.float32)
             + jnp.dot(mx_ref[...], Wo_ref[...],
                       precision=lax.Precision.HIGHEST,
                       preferred_element_type=jnp.float32)
             + bc_ref[...])
        zbuf[rows, :] = z
        ones = jnp.ones((1, M_BLK), jnp.float32)
        ssum[...] += jnp.dot(ones, z, precision=lax.Precision.HIGHEST,
                             preferred_element_type=jnp.float32)

    @pl.when(p == 1)
    def _var():
        zc = zbuf[rows, :] - ssum[...] * (1.0 / BN)
        ones = jnp.ones((1, M_BLK), jnp.float32)
        ssq[...] += jnp.dot(ones, zc * zc, precision=lax.Precision.HIGHEST,
                            preferred_element_type=jnp.float32)

    @pl.when(p == 2)
    def _norm():
        mean = ssum[...] * (1.0 / BN)
        var = ssq[...] * (1.0 / BN)
        inv = g_ref[...] / jnp.sqrt(var + 1e-5)
        out_ref[...] = jnp.maximum((zbuf[rows, :] - mean) * inv + be_ref[...],
                                   0.0)


def _layer(h, mx, A, Wo, bc, g, be):
    return pl.pallas_call(
        _layer_body,
        grid=(3, M_NBLK),
        in_specs=[
            pl.BlockSpec((M_BLK, CP),
                         lambda p, j: (jnp.where(p == 0, j, 0), 0)),
            pl.BlockSpec((M_BLK, CP),
                         lambda p, j: (jnp.where(p == 0, j, 0), 0)),
            pl.BlockSpec((CP, CP), lambda p, j: (0, 0)),
            pl.BlockSpec((CP, CP), lambda p, j: (0, 0)),
            pl.BlockSpec((1, CP), lambda p, j: (0, 0)),
            pl.BlockSpec((1, CP), lambda p, j: (0, 0)),
            pl.BlockSpec((1, CP), lambda p, j: (0, 0)),
        ],
        out_specs=pl.BlockSpec((M_BLK, CP),
                               lambda p, j: (jnp.where(p == 2, j, 0), 0)),
        out_shape=jax.ShapeDtypeStruct((BN, CP), jnp.float32),
        scratch_shapes=[
            pltpu.VMEM((BN, CP), jnp.float32),
            pltpu.VMEM((1, CP), jnp.float32),
            pltpu.VMEM((1, CP), jnp.float32),
        ],
    )(h, mx, A, Wo, bc, g, be)


def _final_body(h_ref, mx_ref, A_ref, Wo_ref, bc_ref, g_ref, be_ref,
                W1_ref, b1_ref, W2_ref, b2_ref, out_ref, zbuf, ssum, ssq):
    p = pl.program_id(0)
    j = pl.program_id(1)
    rows = pl.ds(j * M_BLK, M_BLK)

    @pl.when((p == 0) & (j == 0))
    def _init():
        ssum[...] = jnp.zeros_like(ssum)
        ssq[...] = jnp.zeros_like(ssq)

    @pl.when(p == 0)
    def _compute():
        z = (jnp.dot(h_ref[...], A_ref[...], precision=lax.Precision.HIGHEST,
                     preferred_element_type=jnp.float32)
             + jnp.dot(mx_ref[...], Wo_ref[...],
                       precision=lax.Precision.HIGHEST,
                       preferred_element_type=jnp.float32)
             + bc_ref[...])
        zbuf[rows, :] = z
        ones = jnp.ones((1, M_BLK), jnp.float32)
        ssum[...] += jnp.dot(ones, z, precision=lax.Precision.HIGHEST,
                             preferred_element_type=jnp.float32)

    @pl.when(p == 1)
    def _var():
        zc = zbuf[rows, :] - ssum[...] * (1.0 / BN)
        ones = jnp.ones((1, M_BLK), jnp.float32)
        ssq[...] += jnp.dot(ones, zc * zc, precision=lax.Precision.HIGHEST,
                            preferred_element_type=jnp.float32)

    @pl.when(p == 2)
    def _norm():
        mean = ssum[...] * (1.0 / BN)
        var = ssq[...] * (1.0 / BN)
        inv = g_ref[...] / jnp.sqrt(var + 1e-5)
        hn = jnp.maximum((zbuf[rows, :] - mean) * inv + be_ref[...], 0.0)
        hid = jnp.maximum(
            jnp.dot(hn, W1_ref[...], precision=lax.Precision.HIGHEST,
                    preferred_element_type=jnp.float32)
            + b1_ref[...], 0.0)
        out_ref[...] = (jnp.dot(hid, W2_ref[...], precision=lax.Precision.HIGHEST,
                                preferred_element_type=jnp.float32)
                        + b2_ref[...])


def _final(h, mx, A, Wo, bc, g, be, W1p, b1, W2, b2):
    C2 = W1p.shape[1]
    NC = W2.shape[1]
    return pl.pallas_call(
        _final_body,
        grid=(3, M_NBLK),
        in_specs=[
            pl.BlockSpec((M_BLK, CP),
                         lambda p, j: (jnp.where(p == 0, j, 0), 0)),
            pl.BlockSpec((M_BLK, CP),
                         lambda p, j: (jnp.where(p == 0, j, 0), 0)),
            pl.BlockSpec((CP, CP), lambda p, j: (0, 0)),
            pl.BlockSpec((CP, CP), lambda p, j: (0, 0)),
            pl.BlockSpec((1, CP), lambda p, j: (0, 0)),
            pl.BlockSpec((1, CP), lambda p, j: (0, 0)),
            pl.BlockSpec((1, CP), lambda p, j: (0, 0)),
            pl.BlockSpec((CP, C2), lambda p, j: (0, 0)),
            pl.BlockSpec((1, C2), lambda p, j: (0, 0)),
            pl.BlockSpec((C2, NC), lambda p, j: (0, 0)),
            pl.BlockSpec((1, NC), lambda p, j: (0, 0)),
        ],
        out_specs=pl.BlockSpec((M_BLK, NC),
                               lambda p, j: (jnp.where(p == 2, j, 0), 0)),
        out_shape=jax.ShapeDtypeStruct((BN, NC), jnp.float32),
        scratch_shapes=[
            pltpu.VMEM((BN, CP), jnp.float32),
            pltpu.VMEM((1, CP), jnp.float32),
            pltpu.VMEM((1, CP), jnp.float32),
        ],
    )(h, mx, A, Wo, bc, g, be, W1p, b1, W2, b2)


def _prep_params(Wc, bc, g, be):
    We = Wc[0::2, :]
    Wo = Wc[1::2, :]
    A = We - Wo
    pad2 = ((0, CP - C), (0, CP - C))
    pad1 = ((0, 0), (0, CP - C))
    Ap = jnp.pad(A, pad2)
    Wop = jnp.pad(Wo, pad2)
    bcp = jnp.pad(bc.reshape(1, C), pad1)
    gp = jnp.pad(g.reshape(1, C), pad1)
    bep = jnp.pad(be.reshape(1, C), pad1)
    return Ap, Wop, bcp, gp, bep


def kernel(x, pos, Wc0, bc0, g0, be0, Wc1, bc1, g1, be1, Wc2, bc2, g2, be2,
           W1, b1, W2, b2):
    coords = pos.reshape(B, N, 2)
    coordsT = coords.transpose(0, 2, 1)
    idx8 = _knn(coords, coordsT)
    gidx = (idx8[..., :K].reshape(BN, K)
            .reshape(SC_W, NCH, CHN, K).transpose(0, 1, 3, 2))

    h = x.reshape(B, C, N).transpose(0, 2, 1).reshape(BN, C)
    h = jnp.pad(h, ((0, 0), (0, CP - C)))

    params = [
        (Wc0, bc0, g0, be0),
        (Wc1, bc1, g1, be1),
        (Wc2, bc2, g2, be2),
    ]
    for i, (Wc, bc, g, be) in enumerate(params):
        Ap, Wop, bcp, gp, bep = _prep_params(Wc, bc, g, be)
        mx = _gather_max(h, gidx)
        if i < 2:
            h = _layer(h, mx, Ap, Wop, bcp, gp, bep)
        else:
            W1p = jnp.pad(W1, ((0, CP - C), (0, 0)))
            pred = _final(h, mx, Ap, Wop, bcp, gp, bep,
                          W1p, b1.reshape(1, -1), W2, b2.reshape(1, -1))
    return pred


# speedup vs baseline: 26.2009x; 1.1521x over previous
import functools

import jax
import jax.numpy as jnp
from jax import lax
from jax.experimental import pallas as pl
from jax.experimental.pallas import tpu as pltpu
from jax.experimental.pallas import tpu_sc as plsc

B, C, H, W = 4, 192, 56, 56
N = H * W
BN = B * N
K = 5
CP = 256

KNN_BLK = 392
KNN_NBLK = N // KNN_BLK

SC_W = 32
NPW = BN // SC_W
CHN = 56
NCH = NPW // CHN

M_BLK = 1568
M_NBLK = BN // M_BLK


def _knn_body(coords_ref, coordsT_ref, out_ref):
    b = pl.program_id(0)
    cb = coords_ref[0]
    ct = coordsT_ref[0]
    xi = cb[:, 0:1]
    yi = cb[:, 1:2]
    xj = ct[0:1, :]
    yj = ct[1:2, :]
    dot = jnp.dot(cb, ct, preferred_element_type=jnp.float32)
    sq_i = xi * xi + yi * yi
    sq_j = xj * xj + yj * yj
    d = (sq_i + sq_j) - 2.0 * dot
    lane = lax.broadcasted_iota(jnp.int32, (KNN_BLK, N), 1)
    cols = []
    for _ in range(K):
        mn = jnp.min(d, axis=1, keepdims=True)
        idx = jnp.min(jnp.where(d == mn, lane, N), axis=1, keepdims=True)
        cols.append(idx + b * N)
        d = jnp.where(lane == idx, jnp.inf, d)
    cols.append(jnp.zeros((KNN_BLK, 3), jnp.int32))
    out_ref[0] = jnp.concatenate(cols, axis=1)


def _knn(coords, coordsT):
    return pl.pallas_call(
        _knn_body,
        grid=(B, KNN_NBLK),
        in_specs=[
            pl.BlockSpec((1, KNN_BLK, 2), lambda b, j: (b, j, 0)),
            pl.BlockSpec((1, 2, N), lambda b, j: (b, 0, 0)),
        ],
        out_specs=pl.BlockSpec((1, KNN_BLK, 8), lambda b, j: (b, j, 0)),
        out_shape=jax.ShapeDtypeStruct((B, N, 8), jnp.int32),
    )(coords, coordsT)


def _gather_max(h, gidx):
    mesh = plsc.VectorSubcoreMesh(core_axis_name="c", subcore_axis_name="s")

    @functools.partial(
        pl.kernel,
        out_type=jax.ShapeDtypeStruct((BN, CP), jnp.float32),
        mesh=mesh,
        scratch_types=[
            pltpu.VMEM((K, CHN, CP), jnp.float32),
            pltpu.VMEM((K, CHN), jnp.int32),
            pltpu.VMEM((CHN, CP), jnp.float32),
            pltpu.SemaphoreType.DMA,
        ],
    )
    def run(h_hbm, idx_hbm, out_hbm, rows_v, idx_v, out_v, sem):
        wid = lax.axis_index("s") * 2 + lax.axis_index("c")
        for ch in range(NCH):
            pltpu.sync_copy(idx_hbm.at[wid, ch], idx_v)
            cps = [
                pltpu.async_copy(h_hbm.at[idx_v.at[k]], rows_v.at[k], sem)
                for k in range(K)
            ]
            for cp in cps:
                cp.wait()

            def body(n, carry):
                for c0 in range(CP // 16):
                    sl = pl.ds(c0 * 16, 16)
                    m = rows_v[0, n, sl]
                    for k in range(1, K):
                        m = jnp.maximum(m, rows_v[k, n, sl])
                    out_v[n, sl] = m
                return carry

            lax.fori_loop(0, CHN, body, 0)
            pltpu.sync_copy(
                out_v, out_hbm.at[pl.ds(wid * NPW + ch * CHN, CHN)]
            )

    return run(h, gidx)


def _layer_body(h_ref, mx_ref, A_ref, Wo_ref, bc_ref, g_ref, be_ref,
                out_ref, zbuf, ssum, ssq):
    p = pl.program_id(0)
    j = pl.program_id(1)
    rows = pl.ds(j * M_BLK, M_BLK)

    @pl.when((p == 0) & (j == 0))
    def _init():
        ssum[...] = jnp.zeros_like(ssum)
        ssq[...] = jnp.zeros_like(ssq)

    @pl.when(p == 0)
    def _compute():
        rel = mx_ref[...] - h_ref[...]
        z = (jnp.dot(h_ref[...], A_ref[...],
                     preferred_element_type=jnp.float32)
             + jnp.dot(rel, Wo_ref[...],
                       preferred_element_type=jnp.float32)
             + bc_ref[...])
        zbuf[rows, :] = z
        ones = jnp.ones((1, M_BLK), jnp.float32)
        ssum[...] += jnp.dot(ones, z, precision=lax.Precision.HIGHEST,
                             preferred_element_type=jnp.float32)

    @pl.when(p == 1)
    def _var():
        zc = zbuf[rows, :] - ssum[...] * (1.0 / BN)
        ones = jnp.ones((1, M_BLK), jnp.float32)
        ssq[...] += jnp.dot(ones, zc * zc, precision=lax.Precision.HIGHEST,
                            preferred_element_type=jnp.float32)

    @pl.when(p == 2)
    def _norm():
        mean = ssum[...] * (1.0 / BN)
        var = ssq[...] * (1.0 / BN)
        inv = g_ref[...] / jnp.sqrt(var + 1e-5)
        out_ref[...] = jnp.maximum((zbuf[rows, :] - mean) * inv + be_ref[...],
                                   0.0)


def _layer(h, mx, A, Wo, bc, g, be):
    return pl.pallas_call(
        _layer_body,
        grid=(3, M_NBLK),
        in_specs=[
            pl.BlockSpec((M_BLK, CP),
                         lambda p, j: (jnp.where(p == 0, j, 0), 0)),
            pl.BlockSpec((M_BLK, CP),
                         lambda p, j: (jnp.where(p == 0, j, 0), 0)),
            pl.BlockSpec((CP, CP), lambda p, j: (0, 0)),
            pl.BlockSpec((CP, CP), lambda p, j: (0, 0)),
            pl.BlockSpec((1, CP), lambda p, j: (0, 0)),
            pl.BlockSpec((1, CP), lambda p, j: (0, 0)),
            pl.BlockSpec((1, CP), lambda p, j: (0, 0)),
        ],
        out_specs=pl.BlockSpec((M_BLK, CP),
                               lambda p, j: (jnp.where(p == 2, j, 0), 0)),
        out_shape=jax.ShapeDtypeStruct((BN, CP), jnp.float32),
        scratch_shapes=[
            pltpu.VMEM((BN, CP), jnp.float32),
            pltpu.VMEM((1, CP), jnp.float32),
            pltpu.VMEM((1, CP), jnp.float32),
        ],
    )(h, mx, A, Wo, bc, g, be)


def _final_body(h_ref, mx_ref, A_ref, Wo_ref, bc_ref, g_ref, be_ref,
                W1_ref, b1_ref, W2_ref, b2_ref, out_ref, zbuf, ssum, ssq):
    p = pl.program_id(0)
    j = pl.program_id(1)
    rows = pl.ds(j * M_BLK, M_BLK)

    @pl.when((p == 0) & (j == 0))
    def _init():
        ssum[...] = jnp.zeros_like(ssum)
        ssq[...] = jnp.zeros_like(ssq)

    @pl.when(p == 0)
    def _compute():
        rel = mx_ref[...] - h_ref[...]
        z = (jnp.dot(h_ref[...], A_ref[...],
                     preferred_element_type=jnp.float32)
             + jnp.dot(rel, Wo_ref[...],
                       preferred_element_type=jnp.float32)
             + bc_ref[...])
        zbuf[rows, :] = z
        ones = jnp.ones((1, M_BLK), jnp.float32)
        ssum[...] += jnp.dot(ones, z, precision=lax.Precision.HIGHEST,
                             preferred_element_type=jnp.float32)

    @pl.when(p == 1)
    def _var():
        zc = zbuf[rows, :] - ssum[...] * (1.0 / BN)
        ones = jnp.ones((1, M_BLK), jnp.float32)
        ssq[...] += jnp.dot(ones, zc * zc, precision=lax.Precision.HIGHEST,
                            preferred_element_type=jnp.float32)

    @pl.when(p == 2)
    def _norm():
        mean = ssum[...] * (1.0 / BN)
        var = ssq[...] * (1.0 / BN)
        inv = g_ref[...] / jnp.sqrt(var + 1e-5)
        hn = jnp.maximum((zbuf[rows, :] - mean) * inv + be_ref[...], 0.0)
        hid = jnp.maximum(
            jnp.dot(hn, W1_ref[...],
                    preferred_element_type=jnp.float32)
            + b1_ref[...], 0.0)
        out_ref[...] = (jnp.dot(hid, W2_ref[...],
                                preferred_element_type=jnp.float32)
                        + b2_ref[...])


def _final(h, mx, A, Wo, bc, g, be, W1p, b1, W2, b2):
    C2 = W1p.shape[1]
    NC = W2.shape[1]
    return pl.pallas_call(
        _final_body,
        grid=(3, M_NBLK),
        in_specs=[
            pl.BlockSpec((M_BLK, CP),
                         lambda p, j: (jnp.where(p == 0, j, 0), 0)),
            pl.BlockSpec((M_BLK, CP),
                         lambda p, j: (jnp.where(p == 0, j, 0), 0)),
            pl.BlockSpec((CP, CP), lambda p, j: (0, 0)),
            pl.BlockSpec((CP, CP), lambda p, j: (0, 0)),
            pl.BlockSpec((1, CP), lambda p, j: (0, 0)),
            pl.BlockSpec((1, CP), lambda p, j: (0, 0)),
            pl.BlockSpec((1, CP), lambda p, j: (0, 0)),
            pl.BlockSpec((CP, C2), lambda p, j: (0, 0)),
            pl.BlockSpec((1, C2), lambda p, j: (0, 0)),
            pl.BlockSpec((C2, NC), lambda p, j: (0, 0)),
            pl.BlockSpec((1, NC), lambda p, j: (0, 0)),
        ],
        out_specs=pl.BlockSpec((M_BLK, NC),
                               lambda p, j: (jnp.where(p == 2, j, 0), 0)),
        out_shape=jax.ShapeDtypeStruct((BN, NC), jnp.float32),
        scratch_shapes=[
            pltpu.VMEM((BN, CP), jnp.float32),
            pltpu.VMEM((1, CP), jnp.float32),
            pltpu.VMEM((1, CP), jnp.float32),
        ],
    )(h, mx, A, Wo, bc, g, be, W1p, b1, W2, b2)


def _prep_params(Wc, bc, g, be):
    We = Wc[0::2, :]
    Wo = Wc[1::2, :]
    pad2 = ((0, CP - C), (0, CP - C))
    pad1 = ((0, 0), (0, CP - C))
    Ap = jnp.pad(We, pad2)
    Wop = jnp.pad(Wo, pad2)
    bcp = jnp.pad(bc.reshape(1, C), pad1)
    gp = jnp.pad(g.reshape(1, C), pad1)
    bep = jnp.pad(be.reshape(1, C), pad1)
    return Ap, Wop, bcp, gp, bep


def kernel(x, pos, Wc0, bc0, g0, be0, Wc1, bc1, g1, be1, Wc2, bc2, g2, be2,
           W1, b1, W2, b2):
    coords = pos.reshape(B, N, 2)
    coordsT = coords.transpose(0, 2, 1)
    idx8 = _knn(coords, coordsT)
    gidx = (idx8[..., :K].reshape(BN, K)
            .reshape(SC_W, NCH, CHN, K).transpose(0, 1, 3, 2))

    h = x.reshape(B, C, N).transpose(0, 2, 1).reshape(BN, C)
    h = jnp.pad(h, ((0, 0), (0, CP - C)))

    params = [
        (Wc0, bc0, g0, be0),
        (Wc1, bc1, g1, be1),
        (Wc2, bc2, g2, be2),
    ]
    for i, (Wc, bc, g, be) in enumerate(params):
        Ap, Wop, bcp, gp, bep = _prep_params(Wc, bc, g, be)
        mx = _gather_max(h, gidx)
        if i < 2:
            h = _layer(h, mx, Ap, Wop, bcp, gp, bep)
        else:
            W1p = jnp.pad(W1, ((0, CP - C), (0, 0)))
            pred = _final(h, mx, Ap, Wop, bcp, gp, bep,
                          W1p, b1.reshape(1, -1), W2, b2.reshape(1, -1))
    return pred
